# Initial kernel scaffold; baseline (speedup 1.0000x reference)
import jax
import jax.numpy as jnp
from jax.experimental import pallas as pl


def kernel(pos, x, z, Ws_0, We_0, Wr1_0, br1_0, Wr2_0, Wa_0, Ws_1, We_1, Wr1_1, br1_1, Wr2_1, Wa_1, Ws_2, We_2, Wr1_2, br1_2, Wr2_2, Wa_2):
    raise NotImplementedError("write your pallas kernel here")


# speedup vs baseline: 15.8768x; 15.8768x over previous
import functools

import jax
import jax.numpy as jnp
import numpy as np
from jax import lax
from jax.experimental import pallas as pl
from jax.experimental.pallas import tpu as pltpu
from jax.experimental.pallas import tpu_sc as plsc

MAX_RADIUS = 0.073
NUM_BASIS = 10
N_NODES_TOTAL = 10000
NPAD = 10240
K = 64
NW = 32
ROWS_PER_W = NPAD // NW
EDGES_PER_W = NPAD * K // NW
GCHUNK = 128
RI = 64

_R2 = np.float32(MAX_RADIUS * MAX_RADIUS)
_BASIS_V = np.linspace(0.0, MAX_RADIUS, NUM_BASIS + 2)[1:-1].astype(np.float32)
_INV_STEP = np.float32((NUM_BASIS + 1) / MAX_RADIUS)
_C3 = np.float32(np.sqrt(3.0))
_C5 = np.float32(np.sqrt(5.0))
_C15 = np.float32(np.sqrt(15.0))


def _sc_mesh():
    return plsc.VectorSubcoreMesh(core_axis_name="c", subcore_axis_name="s")


def _worker_id():
    return lax.axis_index("s") * 2 + lax.axis_index("c")


def _build_kernel(px_hbm, py_hbm, pz_hbm, idx_hbm, ex_hbm, ey_hbm, ez_hbm,
                  px_v, py_v, pz_v, ri_v, rx_v, ry_v, rz_v,
                  bi_v, bx_v, by_v, bz_v):
    wid = _worker_id()
    pltpu.sync_copy(px_hbm, px_v)
    pltpu.sync_copy(py_hbm, py_v)
    pltpu.sync_copy(pz_hbm, pz_v)
    iota16 = lax.iota(jnp.int32, 16)
    zero16 = jnp.zeros((16,), jnp.float32)

    def row_body(r, carry):
        i = wid * ROWS_PER_W + r
        iv = jnp.full((16,), i, dtype=jnp.int32)
        for t in range(6):
            sl = pl.ds(t * 16, 16)
            ri_v[sl] = iv
            rx_v[sl] = zero16
            ry_v[sl] = zero16
            rz_v[sl] = zero16
        isl = pl.ds((i // 16) * 16, 16)
        lmask = iota16 == (i % 16)
        pix = jnp.full((16,), jnp.sum(jnp.where(lmask, px_v[isl], 0.0)))
        piy = jnp.full((16,), jnp.sum(jnp.where(lmask, py_v[isl], 0.0)))
        piz = jnp.full((16,), jnp.sum(jnp.where(lmask, pz_v[isl], 0.0)))

        def chunk_body(c, cur):
            sl = pl.ds(c * 16, 16)
            dx = px_v[sl] - pix
            dy = py_v[sl] - piy
            dz = pz_v[sl] - piz
            d2 = dx * dx + dy * dy + dz * dz
            jid = c * 16 + iota16
            m = (d2 < _R2) & (jid != iv)
            cnt = plsc.all_reduce_population_count(m)

            @pl.when(jnp.any(m))
            def _():
                cc = jnp.minimum(cur, K)
                plsc.store_compressed(ri_v.at[pl.ds(cc, 16)], jid, mask=m)
                plsc.store_compressed(rx_v.at[pl.ds(cc, 16)], dx, mask=m)
                plsc.store_compressed(ry_v.at[pl.ds(cc, 16)], dy, mask=m)
                plsc.store_compressed(rz_v.at[pl.ds(cc, 16)], dz, mask=m)

            return cur + cnt[0]

        lax.fori_loop(0, NPAD // 16, chunk_body, jnp.int32(0))
        for t in range(K // 16):
            src = pl.ds(t * 16, 16)
            dst = pl.ds(r * K + t * 16, 16)
            bi_v[dst] = ri_v[src]
            bx_v[dst] = rx_v[src]
            by_v[dst] = ry_v[src]
            bz_v[dst] = rz_v[src]
        return carry

    lax.fori_loop(0, ROWS_PER_W, row_body, jnp.int32(0))
    base = wid * EDGES_PER_W
    pltpu.sync_copy(bi_v, idx_hbm.at[pl.ds(base, EDGES_PER_W)])
    pltpu.sync_copy(bx_v, ex_hbm.at[pl.ds(base, EDGES_PER_W)])
    pltpu.sync_copy(by_v, ey_hbm.at[pl.ds(base, EDGES_PER_W)])
    pltpu.sync_copy(bz_v, ez_hbm.at[pl.ds(base, EDGES_PER_W)])


def _build_lists(px, py, pz):
    out_type = (
        jax.ShapeDtypeStruct((NPAD * K,), jnp.int32),
        jax.ShapeDtypeStruct((NPAD * K,), jnp.float32),
        jax.ShapeDtypeStruct((NPAD * K,), jnp.float32),
        jax.ShapeDtypeStruct((NPAD * K,), jnp.float32),
    )
    scratch = [
        pltpu.VMEM((NPAD,), jnp.float32),
        pltpu.VMEM((NPAD,), jnp.float32),
        pltpu.VMEM((NPAD,), jnp.float32),
        pltpu.VMEM((96,), jnp.int32),
        pltpu.VMEM((96,), jnp.float32),
        pltpu.VMEM((96,), jnp.float32),
        pltpu.VMEM((96,), jnp.float32),
        pltpu.VMEM((EDGES_PER_W,), jnp.int32),
        pltpu.VMEM((EDGES_PER_W,), jnp.float32),
        pltpu.VMEM((EDGES_PER_W,), jnp.float32),
        pltpu.VMEM((EDGES_PER_W,), jnp.float32),
    ]
    fn = pl.kernel(_build_kernel, out_type=out_type, mesh=_sc_mesh(),
                   scratch_types=scratch,
                   compiler_params=pltpu.CompilerParams(
                       needs_layout_passes=False))
    return fn(px, py, pz)


def _gather_kernel(table_hbm, idx_hbm, out_hbm, idx_v, rows_v, sem, *, d):
    wid = _worker_id()
    base = wid * EDGES_PER_W

    def chunk_body(t, carry):
        o = base + t * GCHUNK
        pltpu.sync_copy(idx_hbm.at[pl.ds(o, GCHUNK)], idx_v)
        pltpu.async_copy(table_hbm.at[idx_v], rows_v, sem).wait()
        pltpu.sync_copy(rows_v, out_hbm.at[pl.ds(o, GCHUNK)])
        return carry

    lax.fori_loop(0, EDGES_PER_W // GCHUNK, chunk_body, jnp.int32(0))


def _gather_rows(table, idx_flat):
    d = table.shape[1]
    fn = pl.kernel(
        functools.partial(_gather_kernel, d=d),
        out_type=jax.ShapeDtypeStruct((NPAD * K, d), jnp.float32),
        mesh=_sc_mesh(),
        scratch_types=[
            pltpu.VMEM((GCHUNK,), jnp.int32),
            pltpu.VMEM((GCHUNK, d), jnp.float32),
            pltpu.SemaphoreType.DMA,
        ],
        compiler_params=pltpu.CompilerParams(needs_layout_passes=False,
                                             use_tc_tiling_on_sc=False),
    )
    return fn(table, idx_flat)


def _combine_kernel(g_ref, ex_ref, ey_ref, ez_ref, z_ref,
                    Wr1_ref, br1_ref, Wr2_ref, We_ref, Wa_ref, Wsn_ref,
                    out_ref, *, do, is_final):
    ex = ex_ref[...]
    ey = ey_ref[...]
    ez = ez_ref[...]
    d2 = ex * ex + ey * ey + ez * ez
    elen = jnp.sqrt(d2)
    inv = 1.0 / jnp.maximum(elen, 1e-12)
    ux = ex * inv
    uy = ey * inv
    uz = ez * inv
    sh = jnp.concatenate([
        jnp.ones_like(ux), _C3 * ux, _C3 * uy, _C3 * uz,
        _C15 * ux * uy, _C15 * uy * uz, 0.5 * _C5 * (3.0 * uz * uz - 1.0),
        _C15 * ux * uz, 0.5 * _C15 * (ux * ux - uy * uy)], axis=1)
    kplus1 = (lax.broadcasted_iota(jnp.int32, (1, NUM_BASIS), 1)
              .astype(jnp.float32) + 1.0)
    diff = elen * _INV_STEP - kplus1
    win = ((diff > -1.0) & (diff < 1.0)).astype(jnp.float32)
    emb = jnp.cos((0.5 * np.pi) * diff) * win * np.float32(NUM_BASIS ** 0.5)
    hidden = jnp.maximum(
        jnp.dot(emb, Wr1_ref[...], preferred_element_type=jnp.float32)
        + br1_ref[...], 0.0)
    radial = jnp.dot(hidden, Wr2_ref[...],
                     preferred_element_type=jnp.float32)
    shwe = jnp.dot(sh, We_ref[...],
                   preferred_element_type=jnp.float32)
    msg = (g_ref[...] + shwe) * radial
    acc = jnp.sum(msg.reshape(RI, K, do), axis=1)
    out = acc * np.float32(0.25) + z_ref[...] * Wa_ref[...]
    if is_final:
        part = jnp.sum(out, axis=0, keepdims=True)
        pid = pl.program_id(0)

        @pl.when(pid == 0)
        def _():
            out_ref[...] = part

        @pl.when(pid != 0)
        def _():
            out_ref[...] = out_ref[...] + part
    else:
        h = jnp.tanh(out)
        out_ref[...] = jnp.dot(h, Wsn_ref[...],
                               preferred_element_type=jnp.float32)


def _combine(g, exf, eyf, ezf, zpad, Wr1, br1, Wr2, We, Wa, Wsn, is_final):
    do = g.shape[1]
    dn = Wsn.shape[1]
    p = RI * K
    grid = (NPAD // RI,)
    in_specs = [
        pl.BlockSpec((p, do), lambda t: (t, 0)),
        pl.BlockSpec((p, 1), lambda t: (t, 0)),
        pl.BlockSpec((p, 1), lambda t: (t, 0)),
        pl.BlockSpec((p, 1), lambda t: (t, 0)),
        pl.BlockSpec((RI, 1), lambda t: (t, 0)),
        pl.BlockSpec((NUM_BASIS, 100), lambda t: (0, 0)),
        pl.BlockSpec((1, 100), lambda t: (0, 0)),
        pl.BlockSpec((100, do), lambda t: (0, 0)),
        pl.BlockSpec((9, do), lambda t: (0, 0)),
        pl.BlockSpec((1, do), lambda t: (0, 0)),
        pl.BlockSpec((do, dn), lambda t: (0, 0)),
    ]
    if is_final:
        out_spec = pl.BlockSpec((1, do), lambda t: (0, 0))
        out_shape = jax.ShapeDtypeStruct((1, do), jnp.float32)
    else:
        out_spec = pl.BlockSpec((RI, dn), lambda t: (t, 0))
        out_shape = jax.ShapeDtypeStruct((NPAD, dn), jnp.float32)
    fn = pl.pallas_call(
        functools.partial(_combine_kernel, do=do, is_final=is_final),
        grid=grid, in_specs=in_specs, out_specs=out_spec, out_shape=out_shape)
    return fn(g, exf, eyf, ezf, zpad, Wr1, br1.reshape(1, 100), Wr2, We, Wa, Wsn)


def _matmul_kernel(x_ref, w_ref, o_ref):
    o_ref[...] = jnp.dot(x_ref[...], w_ref[...],
                         preferred_element_type=jnp.float32)


def _matmul(x, w):
    m, kdim = x.shape
    n = w.shape[1]
    bm = 512
    fn = pl.pallas_call(
        _matmul_kernel, grid=(m // bm,),
        in_specs=[pl.BlockSpec((bm, kdim), lambda t: (t, 0)),
                  pl.BlockSpec((kdim, n), lambda t: (0, 0))],
        out_specs=pl.BlockSpec((bm, n), lambda t: (t, 0)),
        out_shape=jax.ShapeDtypeStruct((m, n), jnp.float32))
    return fn(x, w)


def kernel(pos, x, z, Ws_0, We_0, Wr1_0, br1_0, Wr2_0, Wa_0,
           Ws_1, We_1, Wr1_1, br1_1, Wr2_1, Wa_1,
           Ws_2, We_2, Wr1_2, br1_2, Wr2_2, Wa_2):
    n = pos.shape[0]
    pad_n = NPAD - n
    pos_p = jnp.pad(pos, ((0, pad_n), (0, 0)), constant_values=100.0)
    px = pos_p[:, 0].astype(jnp.float32)
    py = pos_p[:, 1].astype(jnp.float32)
    pz = pos_p[:, 2].astype(jnp.float32)
    x_p = jnp.pad(x, ((0, pad_n), (0, 0)))
    z_p = jnp.pad(z, ((0, pad_n), (0, 0)))

    idx_f, ex_f, ey_f, ez_f = _build_lists(px, py, pz)
    ex2 = ex_f.reshape(-1, 1)
    ey2 = ey_f.reshape(-1, 1)
    ez2 = ez_f.reshape(-1, 1)

    d2p = 16
    Ws_2p = jnp.pad(Ws_2, ((0, 0), (0, d2p - 1)))
    We_2p = jnp.pad(We_2, ((0, 0), (0, d2p - 1)))
    Wr2_2p = jnp.pad(Wr2_2, ((0, 0), (0, d2p - 1)))
    Wa_2p = jnp.pad(Wa_2, ((0, 0), (0, d2p - 1)))

    hh0 = _matmul(x_p, Ws_0)
    g0 = _gather_rows(hh0, idx_f)
    hh1 = _combine(g0, ex2, ey2, ez2, z_p, Wr1_0, br1_0, Wr2_0, We_0, Wa_0,
                   Ws_1, is_final=False)
    g1 = _gather_rows(hh1, idx_f)
    hh2 = _combine(g1, ex2, ey2, ez2, z_p, Wr1_1, br1_1, Wr2_1, We_1, Wa_1,
                   Ws_2p, is_final=False)
    g2 = _gather_rows(hh2, idx_f)
    fin = _combine(g2, ex2, ey2, ez2, z_p, Wr1_2, br1_2, Wr2_2p, We_2p, Wa_2p,
                   jnp.zeros((d2p, d2p), jnp.float32), is_final=True)
    return fin[0:1, 0:1] * np.float32(1.0 / np.sqrt(N_NODES_TOTAL))
